# baseline (speedup 1.0000x reference)
import functools

import jax
import jax.numpy as jnp
from jax import lax
from jax.experimental import pallas as pl
from jax.experimental.pallas import tpu as pltpu
from jax.experimental.pallas import tpu_sc as plsc

_VOCAB = 1000000
_DIM = 32
_B = 16384
_C = 20

_NC = 2
_NS = 16
_NW = _NC * _NS
_BPW = _B // _NW
_CHUNK = 32
_NCHUNK = _BPW // _CHUNK
_CTX_PER_CHUNK = _CHUNK * _C
_IDXW = 128
_CIDX_ROWS = _BPW // _IDXW
_XIDX_ROWS = (_BPW * _C) // _IDXW
_XGPC = _CTX_PER_CHUNK // _IDXW

_LN2 = 0.6931471805599453


def _vlog(x):
  bits = plsc.bitcast(x, jnp.int32)
  e = (bits >> 23) - 127
  m = plsc.bitcast((bits & 0x7FFFFF) | 0x3F800000, jnp.float32)
  t = (m - 1.0) / (m + 1.0)
  t2 = t * t
  p = jnp.float32(1.0 / 11.0)
  p = p * t2 + jnp.float32(1.0 / 9.0)
  p = p * t2 + jnp.float32(1.0 / 7.0)
  p = p * t2 + jnp.float32(1.0 / 5.0)
  p = p * t2 + jnp.float32(1.0 / 3.0)
  p = p * t2 + jnp.float32(1.0)
  return e.astype(jnp.float32) * jnp.float32(_LN2) + 2.0 * t * p


def _body(cidx_hbm, xidx_hbm, target_hbm, context_hbm, out_hbm,
          cidx_v, cdma_v, xidx_v, xdma_v, cbuf, xbuf, svals, ovec,
          sem_c, sem_x):
  wid = lax.axis_index("s") * _NC + lax.axis_index("c")

  pltpu.sync_copy(cidx_hbm.at[pl.ds(wid * _CIDX_ROWS, _CIDX_ROWS)], cidx_v)
  pltpu.sync_copy(xidx_hbm.at[pl.ds(wid * _XIDX_ROWS, _XIDX_ROWS)], xidx_v)

  def shift_body(r, carry):
    for k in range(8):
      v = cidx_v[r, pl.ds(k * 16, 16)]
      cdma_v[r, pl.ds(k * 16, 16)] = v >> 2
      cidx_v[r, pl.ds(k * 16, 16)] = (v & 3) * 32
    return carry

  lax.fori_loop(0, _CIDX_ROWS, shift_body, jnp.int32(0))

  def shift_body_x(r, carry):
    for k in range(8):
      v = xidx_v[r, pl.ds(k * 16, 16)]
      xdma_v[r, pl.ds(k * 16, 16)] = v >> 2
      xidx_v[r, pl.ds(k * 16, 16)] = (v & 3) * 32
    return carry

  lax.fori_loop(0, _XIDX_ROWS, shift_body_x, jnp.int32(0))

  lane = lax.iota(jnp.int32, 16)
  lane16 = lane + 16
  tail_mask = lane < (_C - 16)
  zeros = jnp.zeros((16,), jnp.float32)

  def chunk_body(j, t_acc):
    ccopy = pltpu.make_async_copy(
        target_hbm.at[cdma_v.at[j >> 2, pl.ds((j & 3) * _CHUNK, _CHUNK)]],
        cbuf, sem_c)
    ccopy.start()
    for k in range(_XGPC):
      pltpu.make_async_copy(
          context_hbm.at[xdma_v.at[j * _XGPC + k]],
          xbuf.at[pl.ds(k * _IDXW, _IDXW)], sem_x).start()
    ccopy.wait()
    for k in range(_XGPC):
      pltpu.make_async_copy(
          context_hbm.at[xdma_v.at[j * _XGPC + k]],
          xbuf.at[pl.ds(k * _IDXW, _IDXW)], sem_x).wait()

    def row_body(r, t_acc):
      row = j * _CHUNK + r
      cv = plsc.load_gather(
          cidx_v, [jnp.broadcast_to(row >> 7, (16,)),
                   jnp.broadcast_to(row & 127, (16,))])
      rv = jnp.broadcast_to(r, (16,))
      c0 = plsc.load_gather(cbuf, [rv, cv + lane])
      c1 = plsc.load_gather(cbuf, [rv, cv + lane16])
      l1 = zeros
      l2 = zeros
      for c in range(_C):
        p = row * _C + c
        xv = plsc.load_gather(
            xidx_v, [jnp.broadcast_to(p >> 7, (16,)),
                     jnp.broadcast_to(p & 127, (16,))])
        rcv = jnp.broadcast_to(r * _C + c, (16,))
        x0 = plsc.load_gather(xbuf, [rcv, xv + lane])
        x1 = plsc.load_gather(xbuf, [rcv, xv + lane16])
        s = jnp.sum(c0 * x0 + c1 * x1)
        if c < 16:
          l1 = jnp.where(lane == c, s, l1)
        else:
          l2 = jnp.where(lane == c - 16, s, l2)
      e = jnp.exp(l1) + jnp.where(tail_mask, jnp.exp(l2), 0.0)
      plsc.store_scatter(svals, [jnp.broadcast_to(row, (16,))],
                         jnp.broadcast_to(jnp.sum(e), (16,)),
                         mask=lane == 0)
      return t_acc + l1 + jnp.where(tail_mask, l2, zeros)

    return lax.fori_loop(0, _CHUNK, row_body, t_acc)

  t_acc = lax.fori_loop(0, _NCHUNK, chunk_body, zeros)

  def log_body(g, acc):
    return acc + _vlog(svals[pl.ds(g * 16, 16)])

  lse_acc = lax.fori_loop(0, _BPW // 16, log_body, zeros)

  p_lse = jnp.sum(lse_acc)
  p_t = jnp.sum(t_acc)
  ovec[pl.ds(0, 16)] = jnp.where(lane == 0, p_lse,
                                 jnp.where(lane == 1, p_t, 0.0))
  pltpu.sync_copy(ovec, out_hbm.at[wid])


@jax.jit
def _block2vec_partials(cidx, xidx, target_packed, context_packed):
  mesh = plsc.VectorSubcoreMesh(core_axis_name="c", subcore_axis_name="s")
  f = pl.kernel(
      _body,
      out_type=jax.ShapeDtypeStruct((_NW, 128), jnp.float32),
      mesh=mesh,
      compiler_params=pltpu.CompilerParams(
          needs_layout_passes=False, use_tc_tiling_on_sc=True),
      scratch_types=[
          pltpu.VMEM((_CIDX_ROWS, _IDXW), jnp.int32),
          pltpu.VMEM((_CIDX_ROWS, _IDXW), jnp.int32),
          pltpu.VMEM((_XIDX_ROWS, _IDXW), jnp.int32),
          pltpu.VMEM((_XIDX_ROWS, _IDXW), jnp.int32),
          pltpu.VMEM((_CHUNK, _IDXW), jnp.float32),
          pltpu.VMEM((_CTX_PER_CHUNK, _IDXW), jnp.float32),
          pltpu.VMEM((_BPW,), jnp.float32),
          pltpu.VMEM((_IDXW,), jnp.float32),
          pltpu.SemaphoreType.DMA,
          pltpu.SemaphoreType.DMA,
      ],
  )
  return f(cidx, xidx, target_packed, context_packed)


def kernel(center_tiles, context_tiles, target_table, context_table):
  cidx = center_tiles.astype(jnp.int32).reshape(_B // _IDXW, _IDXW)
  xidx = context_tiles.astype(jnp.int32).reshape((_B * _C) // _IDXW, _IDXW)
  tpk = target_table.reshape(_VOCAB * _DIM // _IDXW, _IDXW)
  cpk = context_table.reshape(_VOCAB * _DIM // _IDXW, _IDXW)
  partials = _block2vec_partials(cidx, xidx, tpk, cpk)
  sum_lse = jnp.sum(partials[:, 0])
  sum_logits = jnp.sum(partials[:, 1])
  return sum_lse / _B - sum_logits / (_B * _C)


# speedup vs baseline: 1.0897x; 1.0897x over previous
import functools

import jax
import jax.numpy as jnp
from jax import lax
from jax.experimental import pallas as pl
from jax.experimental.pallas import tpu as pltpu
from jax.experimental.pallas import tpu_sc as plsc

_VOCAB = 1000000
_DIM = 32
_B = 16384
_C = 20

_NC = 2
_NS = 16
_NW = _NC * _NS
_BPW = _B // _NW
_CHUNK = 64
_NCHUNK = _BPW // _CHUNK
_CTX_PER_CHUNK = _CHUNK * _C
_IDXW = 128
_CIDX_ROWS = _BPW // _IDXW
_XIDX_ROWS = (_BPW * _C) // _IDXW
_XROWS_PER_CHUNK = _CTX_PER_CHUNK // _IDXW

_LN2 = 0.6931471805599453


def _vlog(x):
  bits = plsc.bitcast(x, jnp.int32)
  e = (bits >> 23) - 127
  m = plsc.bitcast((bits & 0x7FFFFF) | 0x3F800000, jnp.float32)
  t = (m - 1.0) / (m + 1.0)
  t2 = t * t
  p = jnp.float32(1.0 / 11.0)
  p = p * t2 + jnp.float32(1.0 / 9.0)
  p = p * t2 + jnp.float32(1.0 / 7.0)
  p = p * t2 + jnp.float32(1.0 / 5.0)
  p = p * t2 + jnp.float32(1.0 / 3.0)
  p = p * t2 + jnp.float32(1.0)
  return e.astype(jnp.float32) * jnp.float32(_LN2) + 2.0 * t * p


def _body(cidx_hbm, xidx_hbm, target_hbm, context_hbm, out_hbm,
          cidx_v, xidx_v, crows, xbuf0, xbuf1, svals, ovec,
          sem_c, sem_x0, sem_x1):
  wid = lax.axis_index("s") * _NC + lax.axis_index("c")

  pltpu.sync_copy(cidx_hbm.at[pl.ds(wid * _CIDX_ROWS, _CIDX_ROWS)], cidx_v)
  pltpu.sync_copy(xidx_hbm.at[pl.ds(wid * _XIDX_ROWS, _XIDX_ROWS)], xidx_v)

  for r in range(_CIDX_ROWS):
    pltpu.make_async_copy(
        target_hbm.at[cidx_v.at[r]],
        crows.at[pl.ds(r * _IDXW, _IDXW)], sem_c).start()

  xbufs = (xbuf0, xbuf1)
  sems = (sem_x0, sem_x1)

  def start_ctx(j):
    buf = xbufs[j % 2]
    for k in range(_XROWS_PER_CHUNK):
      pltpu.make_async_copy(
          context_hbm.at[xidx_v.at[j * _XROWS_PER_CHUNK + k]],
          buf.at[pl.ds(k * _IDXW, _IDXW)], sems[j % 2]).start()

  def wait_ctx(j):
    buf = xbufs[j % 2]
    for k in range(_XROWS_PER_CHUNK):
      pltpu.make_async_copy(
          context_hbm.at[xidx_v.at[j * _XROWS_PER_CHUNK + k]],
          buf.at[pl.ds(k * _IDXW, _IDXW)], sems[j % 2]).wait()

  start_ctx(0)
  for r in range(_CIDX_ROWS):
    pltpu.make_async_copy(
        target_hbm.at[cidx_v.at[r]],
        crows.at[pl.ds(r * _IDXW, _IDXW)], sem_c).wait()

  lane = lax.iota(jnp.int32, 16)
  tail_mask = lane < (_C - 16)
  zeros = jnp.zeros((16,), jnp.float32)

  t_acc = zeros
  for j in range(_NCHUNK):
    if j + 1 < _NCHUNK:
      start_ctx(j + 1)
    wait_ctx(j)
    xb = xbufs[j % 2]

    def row_body(r, t_acc, j=j, xb=xb):
      row = j * _CHUNK + r
      c0 = crows[row, pl.ds(0, 16)]
      c1 = crows[row, pl.ds(16, 16)]
      l1 = zeros
      l2 = zeros
      for c in range(_C):
        x0 = xb[r * _C + c, pl.ds(0, 16)]
        x1 = xb[r * _C + c, pl.ds(16, 16)]
        s = jnp.sum(c0 * x0 + c1 * x1)
        if c < 16:
          l1 = jnp.where(lane == c, s, l1)
        else:
          l2 = jnp.where(lane == c - 16, s, l2)
      e = jnp.exp(l1) + jnp.where(tail_mask, jnp.exp(l2), 0.0)
      plsc.store_scatter(svals, [jnp.broadcast_to(row, (16,))],
                         jnp.broadcast_to(jnp.sum(e), (16,)),
                         mask=lane == 0)
      return t_acc + l1 + jnp.where(tail_mask, l2, zeros)

    t_acc = lax.fori_loop(0, _CHUNK, row_body, t_acc)

  def log_body(g, acc):
    return acc + _vlog(svals[pl.ds(g * 16, 16)])

  lse_acc = lax.fori_loop(0, _BPW // 16, log_body, zeros)

  p_lse = jnp.sum(lse_acc)
  p_t = jnp.sum(t_acc)
  ovec[...] = jnp.where(lane == 0, p_lse, jnp.where(lane == 1, p_t, 0.0))
  pltpu.sync_copy(ovec, out_hbm.at[wid])


@jax.jit
def _block2vec_partials(cidx, xidx, target_table, context_table):
  mesh = plsc.VectorSubcoreMesh(core_axis_name="c", subcore_axis_name="s")
  f = pl.kernel(
      _body,
      out_type=jax.ShapeDtypeStruct((_NW, 16), jnp.float32),
      mesh=mesh,
      compiler_params=pltpu.CompilerParams(
          needs_layout_passes=False, use_tc_tiling_on_sc=False),
      scratch_types=[
          pltpu.VMEM((_CIDX_ROWS, _IDXW), jnp.int32),
          pltpu.VMEM((_XIDX_ROWS, _IDXW), jnp.int32),
          pltpu.VMEM((_BPW, _DIM), jnp.float32),
          pltpu.VMEM((_CTX_PER_CHUNK, _DIM), jnp.float32),
          pltpu.VMEM((_CTX_PER_CHUNK, _DIM), jnp.float32),
          pltpu.VMEM((_BPW,), jnp.float32),
          pltpu.VMEM((16,), jnp.float32),
          pltpu.SemaphoreType.DMA,
          pltpu.SemaphoreType.DMA,
          pltpu.SemaphoreType.DMA,
      ],
  )
  return f(cidx, xidx, target_table, context_table)


def kernel(center_tiles, context_tiles, target_table, context_table):
  cidx = center_tiles.astype(jnp.int32).reshape(_B // _IDXW, _IDXW)
  xidx = context_tiles.astype(jnp.int32).reshape((_B * _C) // _IDXW, _IDXW)
  partials = _block2vec_partials(cidx, xidx, target_table, context_table)
  sum_lse = jnp.sum(partials[:, 0])
  sum_logits = jnp.sum(partials[:, 1])
  return sum_lse / _B - sum_logits / (_B * _C)


# speedup vs baseline: 1.4211x; 1.3042x over previous
import functools

import jax
import jax.numpy as jnp
from jax import lax
from jax.experimental import pallas as pl
from jax.experimental.pallas import tpu as pltpu
from jax.experimental.pallas import tpu_sc as plsc

_VOCAB = 1000000
_DIM = 32
_B = 16384
_C = 20

_NC = 2
_NS = 16
_NW = _NC * _NS
_BPW = _B // _NW
_CHUNK = 64
_NCHUNK = _BPW // _CHUNK
_CTX_PER_CHUNK = _CHUNK * _C
_IDXW = 128
_CIDX_ROWS = _BPW // _IDXW
_XIDX_ROWS = (_BPW * _C) // _IDXW
_XROWS_PER_CHUNK = _CTX_PER_CHUNK // _IDXW

_LN2 = 0.6931471805599453


def _vlog(x):
  bits = plsc.bitcast(x, jnp.int32)
  e = (bits >> 23) - 127
  m = plsc.bitcast((bits & 0x7FFFFF) | 0x3F800000, jnp.float32)
  t = (m - 1.0) / (m + 1.0)
  t2 = t * t
  p = jnp.float32(1.0 / 11.0)
  p = p * t2 + jnp.float32(1.0 / 9.0)
  p = p * t2 + jnp.float32(1.0 / 7.0)
  p = p * t2 + jnp.float32(1.0 / 5.0)
  p = p * t2 + jnp.float32(1.0 / 3.0)
  p = p * t2 + jnp.float32(1.0)
  return e.astype(jnp.float32) * jnp.float32(_LN2) + 2.0 * t * p


_NFULL = _VOCAB // _IDXW
_TAIL0 = (_VOCAB // 64) * 64 - 64
_TAILW = _VOCAB - _TAIL0
_PACKR = _VOCAB * _DIM // _IDXW
_SBW = 512
_NSB = _TAIL0 // _SBW
_SBR = _SBW // 4
_NITER = (_NSB // _NW + 1) // 2
_PPITCH = 132


def _pack_body(ttab, ttail, tout, va, vb, oa, ob, opad,
               semi_a, semi_b, semo_a, semo_b):
  wid = lax.axis_index("s") * _NC + lax.axis_index("c")
  lane = lax.iota(jnp.int32, 16)
  rowc = lane >> 2
  colc = (lane & 3) * 33

  def in_copy(sb, buf, sem):
    return pltpu.make_async_copy(
        ttab.at[:, pl.ds(pl.multiple_of(sb * _SBW, _SBW), _SBW)], buf, sem)

  def out_copy(sb, buf, sem):
    return pltpu.make_async_copy(
        buf, tout.at[pl.ds(pl.multiple_of(sb * _SBR, _SBR), _SBR)], sem)

  def transpose(vbuf, obuf):
    def ph1(d, c2):
      cv = colc + jnp.broadcast_to(d, (16,))
      for c in range(_SBW // 16):
        plsc.store_scatter(
            opad, [rowc + 4 * c, cv], vbuf[d, pl.ds(16 * c, 16)])
      return c2

    lax.fori_loop(0, 32, ph1, jnp.int32(0))

    pcols = [lane + (33 * j + 16 * h) for j in range(4) for h in range(2)]

    def ph2(rr, c2):
      rv = jnp.broadcast_to(rr, (16,))
      for t, cvec in enumerate(pcols):
        obuf[rr, pl.ds(16 * t, 16)] = plsc.load_gather(opad, [rv, cvec])
      return c2

    lax.fori_loop(0, _SBR, ph2, jnp.int32(0))

  in_copy(wid, va, semi_a).start()
  in_copy(wid + _NW, vb, semi_b).start()

  def iter_body(i, carry):
    sb_a = wid + 2 * i * _NW
    in_copy(sb_a, va, semi_a).wait()

    @pl.when(i > 0)
    def _wa():
      out_copy(sb_a - 2 * _NW, oa, semo_a).wait()

    transpose(va, oa)
    out_copy(sb_a, oa, semo_a).start()

    @pl.when(i < _NITER - 1)
    def _na():
      in_copy(sb_a + 2 * _NW, va, semi_a).start()

    sb_b = sb_a + _NW

    @pl.when(sb_b < _NSB)
    def _bphase():
      in_copy(sb_b, vb, semi_b).wait()

      @pl.when(i > 0)
      def _wb():
        out_copy(sb_b - 2 * _NW, ob, semo_b).wait()

      transpose(vb, ob)
      out_copy(sb_b, ob, semo_b).start()

      @pl.when(sb_b + 2 * _NW < _NSB)
      def _nb():
        in_copy(sb_b + 2 * _NW, vb, semi_b).start()

    return carry

  lax.fori_loop(0, _NITER, iter_body, jnp.int32(0))

  out_copy(wid + 2 * (_NITER - 1) * _NW, oa, semo_a).wait()
  sb_b_last = jnp.where(wid == 0, _NSB - 1,
                        wid + _NW + 2 * (_NITER - 2) * _NW)
  out_copy(sb_b_last, ob, semo_b).wait()

  @pl.when(wid == _NW - 1)
  def _tail():
    pltpu.sync_copy(ttail, oa.at[pl.ds(0, _TAILW // 4)])
    pltpu.sync_copy(oa.at[pl.ds(0, _TAILW // 4)],
                    tout.at[pl.ds(_PACKR - _TAILW // 4, _TAILW // 4)])


@jax.jit
def _pack_target(ttab_t, ttail):
  mesh = plsc.VectorSubcoreMesh(core_axis_name="c", subcore_axis_name="s")
  f = pl.kernel(
      _pack_body,
      out_type=jax.ShapeDtypeStruct((_PACKR, _IDXW), jnp.float32),
      mesh=mesh,
      compiler_params=pltpu.CompilerParams(
          needs_layout_passes=False, use_tc_tiling_on_sc=True),
      scratch_types=[
          pltpu.VMEM((32, _SBW), jnp.float32),
          pltpu.VMEM((32, _SBW), jnp.float32),
          pltpu.VMEM((_SBR, _IDXW), jnp.float32),
          pltpu.VMEM((_SBR, _IDXW), jnp.float32),
          pltpu.VMEM((_SBR, _PPITCH), jnp.float32),
          pltpu.SemaphoreType.DMA,
          pltpu.SemaphoreType.DMA,
          pltpu.SemaphoreType.DMA,
          pltpu.SemaphoreType.DMA,
      ],
  )
  return f(ttab_t, ttail)


def _body(cidx_hbm, xidx_hbm, target_hbm, context_hbm, out_hbm,
          cidx_v, cdma_v, xidx_v, crows0, crows1, xbuf0, xbuf1, svals, ovec,
          sem_c0, sem_c1, sem_x0, sem_x1):
  wid = lax.axis_index("s") * _NC + lax.axis_index("c")
  lane = lax.iota(jnp.int32, 16)
  lane16 = lane + 16

  pltpu.sync_copy(cidx_hbm.at[pl.ds(wid * _CIDX_ROWS, _CIDX_ROWS)], cidx_v)
  pltpu.sync_copy(xidx_hbm.at[pl.ds(wid * _XIDX_ROWS, _XIDX_ROWS)], xidx_v)

  def shift_body(r, carry):
    for k in range(8):
      v = cidx_v[r, pl.ds(k * 16, 16)]
      cdma_v[r, pl.ds(k * 16, 16)] = v >> 2
      cidx_v[r, pl.ds(k * 16, 16)] = (v & 3) * 32
    return carry

  lax.fori_loop(0, _CIDX_ROWS, shift_body, jnp.int32(0))

  xbufs = (xbuf0, xbuf1)
  sems = (sem_x0, sem_x1)
  cbufs = (crows0, crows1)
  csems = (sem_c0, sem_c1)

  def start_ctx(j):
    buf = xbufs[j % 2]
    for k in range(_XROWS_PER_CHUNK):
      pltpu.make_async_copy(
          context_hbm.at[xidx_v.at[j * _XROWS_PER_CHUNK + k]],
          buf.at[pl.ds(k * _IDXW, _IDXW)], sems[j % 2]).start()

  def wait_ctx(j):
    buf = xbufs[j % 2]
    for k in range(_XROWS_PER_CHUNK):
      pltpu.make_async_copy(
          context_hbm.at[xidx_v.at[j * _XROWS_PER_CHUNK + k]],
          buf.at[pl.ds(k * _IDXW, _IDXW)], sems[j % 2]).wait()

  def ctr_copy(j):
    return pltpu.make_async_copy(
        target_hbm.at[cdma_v.at[j >> 1, pl.ds((j & 1) * _CHUNK, _CHUNK)]],
        cbufs[j % 2], csems[j % 2])

  start_ctx(0)
  ctr_copy(0).start()

  tail_mask = lane < (_C - 16)
  zeros = jnp.zeros((16,), jnp.float32)

  t_acc = zeros
  for j in range(_NCHUNK):
    if j + 1 < _NCHUNK:
      start_ctx(j + 1)
      ctr_copy(j + 1).start()
    wait_ctx(j)
    ctr_copy(j).wait()
    xb = xbufs[j % 2]
    cb = cbufs[j % 2]

    def row_body(r, t_acc, j=j, xb=xb, cb=cb):
      row = j * _CHUNK + r
      cv = plsc.load_gather(
          cidx_v, [jnp.broadcast_to(row >> 7, (16,)),
                   jnp.broadcast_to(row & 127, (16,))])
      rv = jnp.broadcast_to(r, (16,))
      c0 = plsc.load_gather(cb, [rv, cv + lane])
      c1 = plsc.load_gather(cb, [rv, cv + lane16])
      l1 = zeros
      l2 = zeros
      for c in range(_C):
        x0 = xb[r * _C + c, pl.ds(0, 16)]
        x1 = xb[r * _C + c, pl.ds(16, 16)]
        s = jnp.sum(c0 * x0 + c1 * x1)
        if c < 16:
          l1 = jnp.where(lane == c, s, l1)
        else:
          l2 = jnp.where(lane == c - 16, s, l2)
      e = jnp.exp(l1) + jnp.where(tail_mask, jnp.exp(l2), 0.0)
      plsc.store_scatter(svals, [jnp.broadcast_to(row, (16,))],
                         jnp.broadcast_to(jnp.sum(e), (16,)),
                         mask=lane == 0)
      return t_acc + l1 + jnp.where(tail_mask, l2, zeros)

    t_acc = lax.fori_loop(0, _CHUNK, row_body, t_acc)

  def log_body(g, acc):
    return acc + _vlog(svals[pl.ds(g * 16, 16)])

  lse_acc = lax.fori_loop(0, _BPW // 16, log_body, zeros)

  p_lse = jnp.sum(lse_acc)
  p_t = jnp.sum(t_acc)
  ovec[...] = jnp.where(lane == 0, p_lse, jnp.where(lane == 1, p_t, 0.0))
  pltpu.sync_copy(ovec, out_hbm.at[wid])


@jax.jit
def _block2vec_partials(cidx, xidx, target_table, context_table):
  mesh = plsc.VectorSubcoreMesh(core_axis_name="c", subcore_axis_name="s")
  f = pl.kernel(
      _body,
      out_type=jax.ShapeDtypeStruct((_NW, 16), jnp.float32),
      mesh=mesh,
      compiler_params=pltpu.CompilerParams(
          needs_layout_passes=False, use_tc_tiling_on_sc=False),
      scratch_types=[
          pltpu.VMEM((_CIDX_ROWS, _IDXW), jnp.int32),
          pltpu.VMEM((_CIDX_ROWS, _IDXW), jnp.int32),
          pltpu.VMEM((_XIDX_ROWS, _IDXW), jnp.int32),
          pltpu.VMEM((_CHUNK, _IDXW), jnp.float32),
          pltpu.VMEM((_CHUNK, _IDXW), jnp.float32),
          pltpu.VMEM((_CTX_PER_CHUNK, _DIM), jnp.float32),
          pltpu.VMEM((_CTX_PER_CHUNK, _DIM), jnp.float32),
          pltpu.VMEM((_BPW,), jnp.float32),
          pltpu.VMEM((16,), jnp.float32),
          pltpu.SemaphoreType.DMA,
          pltpu.SemaphoreType.DMA,
          pltpu.SemaphoreType.DMA,
          pltpu.SemaphoreType.DMA,
      ],
  )
  return f(cidx, xidx, target_table, context_table)


def kernel(center_tiles, context_tiles, target_table, context_table):
  cidx = center_tiles.astype(jnp.int32).reshape(_B // _IDXW, _IDXW)
  xidx = context_tiles.astype(jnp.int32).reshape((_B * _C) // _IDXW, _IDXW)
  ttail = target_table[_TAIL0:].reshape(_TAILW // 4, _IDXW)
  tpk = _pack_target(target_table.T, ttail)
  partials = _block2vec_partials(cidx, xidx, tpk, context_table)
  sum_lse = jnp.sum(partials[:, 0])
  sum_logits = jnp.sum(partials[:, 1])
  return sum_lse / _B - sum_logits / (_B * _C)


# speedup vs baseline: 1.4287x; 1.0054x over previous
import functools

import jax
import jax.numpy as jnp
from jax import lax
from jax.experimental import pallas as pl
from jax.experimental.pallas import tpu as pltpu
from jax.experimental.pallas import tpu_sc as plsc

_VOCAB = 1000000
_DIM = 32
_B = 16384
_C = 20

_NC = 2
_NS = 16
_NW = _NC * _NS
_BPW = _B // _NW
_CHUNK = 64
_NCHUNK = _BPW // _CHUNK
_CTX_PER_CHUNK = _CHUNK * _C
_IDXW = 128
_CIDX_ROWS = _BPW // _IDXW
_XIDX_ROWS = (_BPW * _C) // _IDXW
_XROWS_PER_CHUNK = _CTX_PER_CHUNK // _IDXW

_LN2 = 0.6931471805599453


def _vlog(x):
  bits = plsc.bitcast(x, jnp.int32)
  e = (bits >> 23) - 127
  m = plsc.bitcast((bits & 0x7FFFFF) | 0x3F800000, jnp.float32)
  t = (m - 1.0) / (m + 1.0)
  t2 = t * t
  p = jnp.float32(1.0 / 11.0)
  p = p * t2 + jnp.float32(1.0 / 9.0)
  p = p * t2 + jnp.float32(1.0 / 7.0)
  p = p * t2 + jnp.float32(1.0 / 5.0)
  p = p * t2 + jnp.float32(1.0 / 3.0)
  p = p * t2 + jnp.float32(1.0)
  return e.astype(jnp.float32) * jnp.float32(_LN2) + 2.0 * t * p


_NFULL = _VOCAB // _IDXW
_TAIL0 = (_VOCAB // 64) * 64 - 64
_TAILW = _VOCAB - _TAIL0
_PACKR = _VOCAB * _DIM // _IDXW
_SBW = 512
_NSB = _TAIL0 // _SBW
_SBR = _SBW // 4
_NITER = (_NSB // _NW + 1) // 2
_PPITCH = 132


def _pack_body(ttab, ttail, tout, va, vb, oa, ob, opad,
               semi_a, semi_b, semo_a, semo_b):
  wid = lax.axis_index("s") * _NC + lax.axis_index("c")
  lane = lax.iota(jnp.int32, 16)
  rowc = lane >> 2
  colc = (lane & 3) * 33

  def in_copy(sb, buf, sem):
    return pltpu.make_async_copy(
        ttab.at[:, pl.ds(pl.multiple_of(sb * _SBW, _SBW), _SBW)], buf, sem)

  def out_copy(sb, buf, sem):
    return pltpu.make_async_copy(
        buf, tout.at[pl.ds(pl.multiple_of(sb * _SBR, _SBR), _SBR)], sem)

  base_sc = rowc * _PPITCH + colc

  def transpose(vbuf, obuf):
    def ph1(d, c2):
      dv = base_sc + jnp.broadcast_to(d, (16,))
      for c in range(_SBW // 16):
        plsc.store_scatter(
            opad, [dv + (4 * _PPITCH) * c], vbuf[d, pl.ds(16 * c, 16)])
      return c2

    lax.fori_loop(0, 32, ph1, jnp.int32(0))

    pcols = [lane + (33 * j + 16 * h) for j in range(4) for h in range(2)]

    def ph2(rr, c2):
      rv = jnp.broadcast_to(rr * _PPITCH, (16,))
      for t, cvec in enumerate(pcols):
        obuf[rr, pl.ds(16 * t, 16)] = plsc.load_gather(opad, [rv + cvec])
      return c2

    lax.fori_loop(0, _SBR, ph2, jnp.int32(0))

  in_copy(wid, va, semi_a).start()
  in_copy(wid + _NW, vb, semi_b).start()

  def iter_body(i, carry):
    sb_a = wid + 2 * i * _NW
    in_copy(sb_a, va, semi_a).wait()

    @pl.when(i > 0)
    def _wa():
      out_copy(sb_a - 2 * _NW, oa, semo_a).wait()

    transpose(va, oa)
    out_copy(sb_a, oa, semo_a).start()

    @pl.when(i < _NITER - 1)
    def _na():
      in_copy(sb_a + 2 * _NW, va, semi_a).start()

    sb_b = sb_a + _NW

    @pl.when(sb_b < _NSB)
    def _bphase():
      in_copy(sb_b, vb, semi_b).wait()

      @pl.when(i > 0)
      def _wb():
        out_copy(sb_b - 2 * _NW, ob, semo_b).wait()

      transpose(vb, ob)
      out_copy(sb_b, ob, semo_b).start()

      @pl.when(sb_b + 2 * _NW < _NSB)
      def _nb():
        in_copy(sb_b + 2 * _NW, vb, semi_b).start()

    return carry

  lax.fori_loop(0, _NITER, iter_body, jnp.int32(0))

  out_copy(wid + 2 * (_NITER - 1) * _NW, oa, semo_a).wait()
  sb_b_last = jnp.where(wid == 0, _NSB - 1,
                        wid + _NW + 2 * (_NITER - 2) * _NW)
  out_copy(sb_b_last, ob, semo_b).wait()

  @pl.when(wid == _NW - 1)
  def _tail():
    pltpu.sync_copy(ttail, oa.at[pl.ds(0, _TAILW // 4)])
    pltpu.sync_copy(oa.at[pl.ds(0, _TAILW // 4)],
                    tout.at[pl.ds(_PACKR - _TAILW // 4, _TAILW // 4)])


@jax.jit
def _pack_target(ttab_t, ttail):
  mesh = plsc.VectorSubcoreMesh(core_axis_name="c", subcore_axis_name="s")
  f = pl.kernel(
      _pack_body,
      out_type=jax.ShapeDtypeStruct((_PACKR, _IDXW), jnp.float32),
      mesh=mesh,
      compiler_params=pltpu.CompilerParams(
          needs_layout_passes=False, use_tc_tiling_on_sc=True),
      scratch_types=[
          pltpu.VMEM((32, _SBW), jnp.float32),
          pltpu.VMEM((32, _SBW), jnp.float32),
          pltpu.VMEM((_SBR, _IDXW), jnp.float32),
          pltpu.VMEM((_SBR, _IDXW), jnp.float32),
          pltpu.VMEM((_SBR * _PPITCH,), jnp.float32),
          pltpu.SemaphoreType.DMA,
          pltpu.SemaphoreType.DMA,
          pltpu.SemaphoreType.DMA,
          pltpu.SemaphoreType.DMA,
      ],
  )
  return f(ttab_t, ttail)


def _body(cidx_hbm, xidx_hbm, target_hbm, context_hbm, out_hbm,
          cidx_v, cdma_v, xidx_v, crows0, crows1, xbuf0, xbuf1, svals, ovec,
          sem_c0, sem_c1, sem_x0, sem_x1):
  wid = lax.axis_index("s") * _NC + lax.axis_index("c")
  lane = lax.iota(jnp.int32, 16)
  lane16 = lane + 16

  pltpu.sync_copy(cidx_hbm.at[pl.ds(wid * _CIDX_ROWS, _CIDX_ROWS)], cidx_v)
  pltpu.sync_copy(xidx_hbm.at[pl.ds(wid * _XIDX_ROWS, _XIDX_ROWS)], xidx_v)

  def shift_body(r, carry):
    for k in range(8):
      v = cidx_v[r, pl.ds(k * 16, 16)]
      cdma_v[r, pl.ds(k * 16, 16)] = v >> 2
      cidx_v[r, pl.ds(k * 16, 16)] = (v & 3) * 32
    return carry

  lax.fori_loop(0, _CIDX_ROWS, shift_body, jnp.int32(0))

  xbufs = (xbuf0, xbuf1)
  sems = (sem_x0, sem_x1)
  cbufs = (crows0, crows1)
  csems = (sem_c0, sem_c1)

  def start_ctx(j):
    buf = xbufs[j % 2]
    for k in range(_XROWS_PER_CHUNK):
      pltpu.make_async_copy(
          context_hbm.at[xidx_v.at[j * _XROWS_PER_CHUNK + k]],
          buf.at[pl.ds(k * _IDXW, _IDXW)], sems[j % 2]).start()

  def wait_ctx(j):
    buf = xbufs[j % 2]
    for k in range(_XROWS_PER_CHUNK):
      pltpu.make_async_copy(
          context_hbm.at[xidx_v.at[j * _XROWS_PER_CHUNK + k]],
          buf.at[pl.ds(k * _IDXW, _IDXW)], sems[j % 2]).wait()

  def ctr_copy(j):
    return pltpu.make_async_copy(
        target_hbm.at[cdma_v.at[j >> 1, pl.ds((j & 1) * _CHUNK, _CHUNK)]],
        cbufs[j % 2], csems[j % 2])

  start_ctx(0)
  ctr_copy(0).start()

  tail_mask = lane < (_C - 16)
  zeros = jnp.zeros((16,), jnp.float32)

  t_acc = zeros
  for j in range(_NCHUNK):
    if j + 1 < _NCHUNK:
      start_ctx(j + 1)
      ctr_copy(j + 1).start()
    wait_ctx(j)
    ctr_copy(j).wait()
    xb = xbufs[j % 2]
    cb = cbufs[j % 2]

    def row_body(r, t_acc, j=j, xb=xb, cb=cb):
      row = j * _CHUNK + r
      cv = plsc.load_gather(
          cidx_v, [jnp.broadcast_to(row >> 7, (16,)),
                   jnp.broadcast_to(row & 127, (16,))])
      rv = jnp.broadcast_to(r, (16,))
      c0 = plsc.load_gather(cb, [rv, cv + lane])
      c1 = plsc.load_gather(cb, [rv, cv + lane16])
      l1 = zeros
      l2 = zeros
      for c in range(_C):
        x0 = xb[r * _C + c, pl.ds(0, 16)]
        x1 = xb[r * _C + c, pl.ds(16, 16)]
        s = jnp.sum(c0 * x0 + c1 * x1)
        if c < 16:
          l1 = jnp.where(lane == c, s, l1)
        else:
          l2 = jnp.where(lane == c - 16, s, l2)
      e = jnp.exp(l1) + jnp.where(tail_mask, jnp.exp(l2), 0.0)
      plsc.store_scatter(svals, [jnp.broadcast_to(row, (16,))],
                         jnp.broadcast_to(jnp.sum(e), (16,)),
                         mask=lane == 0)
      return t_acc + l1 + jnp.where(tail_mask, l2, zeros)

    t_acc = lax.fori_loop(0, _CHUNK, row_body, t_acc)

  def log_body(g, acc):
    return acc + _vlog(svals[pl.ds(g * 16, 16)])

  lse_acc = lax.fori_loop(0, _BPW // 16, log_body, zeros)

  p_lse = jnp.sum(lse_acc)
  p_t = jnp.sum(t_acc)
  ovec[...] = jnp.where(lane == 0, p_lse, jnp.where(lane == 1, p_t, 0.0))
  pltpu.sync_copy(ovec, out_hbm.at[wid])


@jax.jit
def _block2vec_partials(cidx, xidx, target_table, context_table):
  mesh = plsc.VectorSubcoreMesh(core_axis_name="c", subcore_axis_name="s")
  f = pl.kernel(
      _body,
      out_type=jax.ShapeDtypeStruct((_NW, 16), jnp.float32),
      mesh=mesh,
      compiler_params=pltpu.CompilerParams(
          needs_layout_passes=False, use_tc_tiling_on_sc=False),
      scratch_types=[
          pltpu.VMEM((_CIDX_ROWS, _IDXW), jnp.int32),
          pltpu.VMEM((_CIDX_ROWS, _IDXW), jnp.int32),
          pltpu.VMEM((_XIDX_ROWS, _IDXW), jnp.int32),
          pltpu.VMEM((_CHUNK, _IDXW), jnp.float32),
          pltpu.VMEM((_CHUNK, _IDXW), jnp.float32),
          pltpu.VMEM((_CTX_PER_CHUNK, _DIM), jnp.float32),
          pltpu.VMEM((_CTX_PER_CHUNK, _DIM), jnp.float32),
          pltpu.VMEM((_BPW,), jnp.float32),
          pltpu.VMEM((16,), jnp.float32),
          pltpu.SemaphoreType.DMA,
          pltpu.SemaphoreType.DMA,
          pltpu.SemaphoreType.DMA,
          pltpu.SemaphoreType.DMA,
      ],
  )
  return f(cidx, xidx, target_table, context_table)


def kernel(center_tiles, context_tiles, target_table, context_table):
  cidx = center_tiles.astype(jnp.int32).reshape(_B // _IDXW, _IDXW)
  xidx = context_tiles.astype(jnp.int32).reshape((_B * _C) // _IDXW, _IDXW)
  ttail = target_table[_TAIL0:].reshape(_TAILW // 4, _IDXW)
  tpk = _pack_target(target_table.T, ttail)
  partials = _block2vec_partials(cidx, xidx, tpk, context_table)
  sum_lse = jnp.sum(partials[:, 0])
  sum_logits = jnp.sum(partials[:, 1])
  return sum_lse / _B - sum_logits / (_B * _C)
